# baseline (speedup 1.0000x reference)
import jax
import jax.numpy as jnp
from jax.experimental import pallas as pl

_N = 100000
_NG = 400
_T = 100
_NB = 4
_BL = 25000
_GOUT = 32
_GRU_H = 64


def _gcn_conv(x, edge_index, edge_weight, W, b):
    num_nodes = x.shape[0]
    row = edge_index[0]
    col = edge_index[1]
    loop = jnp.arange(num_nodes, dtype=row.dtype)
    row = jnp.concatenate([row, loop])
    col = jnp.concatenate([col, loop])
    ew = jnp.concatenate([edge_weight, jnp.ones((num_nodes,), dtype=edge_weight.dtype)])
    deg = jnp.zeros((num_nodes,), dtype=ew.dtype).at[col].add(ew)
    dinv = jnp.where(deg > 0, 1.0 / jnp.sqrt(deg), 0.0)
    norm = dinv[row] * ew * dinv[col]
    xw = x @ W
    msgs = xw[row] * norm[:, None]
    out = jnp.zeros((num_nodes, W.shape[1]), dtype=xw.dtype).at[col].add(msgs)
    return out + b


def _graph_norm(x, batch, weight, bias, mean_scale, num_graphs=_NG, eps=1e-5):
    ones = jnp.ones((x.shape[0],), dtype=x.dtype)
    counts = jax.ops.segment_sum(ones, batch, num_segments=num_graphs)
    counts = jnp.maximum(counts, 1.0)
    mean = jax.ops.segment_sum(x, batch, num_segments=num_graphs) / counts[:, None]
    sub = x - mean[batch] * mean_scale
    var = jax.ops.segment_sum(sub * sub, batch, num_segments=num_graphs) / counts[:, None]
    std = jnp.sqrt(var + eps)
    return weight * sub / std[batch] + bias


def _gru_layer(seq_tbd, Wih, Whh, bih, bhh):
    B = seq_tbd.shape[1]
    H = Whh.shape[1]
    h0 = jnp.zeros((B, H), dtype=seq_tbd.dtype)

    def step(h, xt):
        gi = xt @ Wih.T + bih
        gh = h @ Whh.T + bhh
        ir, iz, inn = jnp.split(gi, 3, axis=-1)
        hr, hz, hn = jnp.split(gh, 3, axis=-1)
        r = jax.nn.sigmoid(ir + hr)
        z = jax.nn.sigmoid(iz + hz)
        n = jnp.tanh(inn + r * hn)
        h_new = (1.0 - z) * n + z * h
        return h_new, h_new

    _, ys = jax.lax.scan(step, h0, seq_tbd)
    return ys


def _head_kernel(zf_ref, w_ref, b_ref, out_ref):
    out_ref[...] = zf_ref[...] @ w_ref[...] + b_ref[0, 0]


def kernel(x, edge_index, edge_attr, batch, block_lengths, params):
    h = x
    for li in range(5):
        h = _gcn_conv(h, edge_index, edge_attr, params['W%d' % li], params['b%d' % li])
        h = _graph_norm(h, batch, params['gn_w%d' % li], params['gn_b%d' % li], params['gn_ms%d' % li])
        if li < 4:
            h = jax.nn.relu(h)
    n_per = _BL // _T
    Hm = h.reshape(_NB, _T, n_per, _GOUT)
    Hm = jnp.transpose(Hm, (0, 2, 1, 3)).reshape(_NB * n_per, _T, _GOUT)
    seq = jnp.transpose(Hm, (1, 0, 2))
    seq = _gru_layer(seq, params['Wih0'], params['Whh0'], params['bih0'], params['bhh0'])
    seq = _gru_layer(seq, params['Wih1'], params['Whh1'], params['bih1'], params['bhh1'])
    zf = seq[-1]
    out = pl.pallas_call(
        _head_kernel,
        out_shape=jax.ShapeDtypeStruct((zf.shape[0], 1), zf.dtype),
    )(zf, params['Wr'].T, params['br'].reshape(1, 1))
    return out


# speedup vs baseline: 1.2615x; 1.2615x over previous
import functools

import jax
import jax.numpy as jnp
from jax import lax
from jax.experimental import pallas as pl
from jax.experimental.pallas import tpu as pltpu
from jax.experimental.pallas import tpu_sc as plsc

N = 100000
E = 1600000
FIN = 21
HID = 32
GOUT = 32
GRU_H = 64
T = 100
NB = 4
BL = 25000
NG = 400

NC = 2
NS = 16
L = 16
HALF = N // NC
C = 512
NCHUNK = 200
SHARE = C * NCHUNK
EPAD = NS * SHARE
NCHUNK3 = 100
SHARE3 = C * NCHUNK3
XW = 128
GROUPS = C // L
IDXR = C // 128
CS = 128
NCHUNKS = SHARE // CS
GROUPS_S = CS // L
CA = 256
GROUPSA = CA // L
NCHUNKA = 200
SHAREA = CA * NCHUNKA

TBL = 16 * 3136
ZSH = TBL // NS
TBL1 = 16 * 3200
ZSH1 = TBL1 // NS
RS = N // 32
RB = 2000
RB4 = 5000

_mesh = plsc.VectorSubcoreMesh(core_axis_name="c", subcore_axis_name="s")


def _deg_kernel(colh, ewh, degh, tbl, cbuf, ebuf, idx, sem):
    cid = lax.axis_index("c")
    sid = lax.axis_index("s")
    base_col = cid * HALF
    lanes = lax.iota(jnp.int32, L)

    def zbody(i, _):
        ebuf[pl.ds(i * L, L)] = jnp.zeros((L,), jnp.float32)
        return 0
    lax.fori_loop(0, C // L, zbody, 0)
    z0 = sid * ZSH1
    for q in range(0, ZSH1, C):
        sz = min(C, ZSH1 - q)
        pltpu.sync_copy(ebuf.at[pl.ds(0, sz)], tbl.at[pl.ds(z0 + q, sz)])
    plsc.subcore_barrier()

    def chunk(k, _):
        base = sid * SHARE + k * C
        pltpu.sync_copy(colh.at[pl.ds(base, C)], cbuf)
        pltpu.sync_copy(ewh.at[pl.ds(base, C)], ebuf)

        def group(i, _):
            c16 = cbuf[pl.ds(i * L, L)]
            local = c16 - base_col
            ok = (local >= 0) & (local < HALF)
            tgt = jnp.where(ok, local, HALF + lanes)
            j = i // 8
            idx[j, pl.ds((i - j * 8) * L, L)] = tgt
            return 0
        lax.fori_loop(0, GROUPS, group, 0)
        for j in range(IDXR):
            pltpu.sync_copy(ebuf.at[pl.ds(j * 128, 128)],
                            tbl.at[idx.at[j]], add=True)
        return 0
    lax.fori_loop(0, NCHUNK, chunk, 0)
    plsc.subcore_barrier()
    pltpu.sync_copy(tbl.at[pl.ds(z0, ZSH1)], degh.at[cid].at[pl.ds(z0, ZSH1)])


_deg_call = pl.kernel(
    _deg_kernel,
    out_type=jax.ShapeDtypeStruct((NC, TBL1), jnp.float32),
    mesh=_mesh,
    scratch_types=[
        pltpu.VMEM_SHARED((TBL1,), jnp.float32),
        pltpu.VMEM((C,), jnp.int32),
        pltpu.VMEM((C,), jnp.float32),
        pltpu.VMEM((IDXR, 128), jnp.int32),
        pltpu.SemaphoreType.DMA,
    ],
)


def _wcol_kernel(colh, ewh, dinvh, wcolh, cbuf, ebuf, dbuf, sem):
    cid = lax.axis_index("c")
    sid = lax.axis_index("s")
    wid = sid * NC + cid

    def chunk(k, _):
        base = wid * SHARE3 + k * C
        pltpu.sync_copy(colh.at[pl.ds(base, C)], cbuf)
        pltpu.sync_copy(ewh.at[pl.ds(base, C)], ebuf)

        def clamp(i, _):
            c16 = cbuf[pl.ds(i * L, L)]
            cbuf[pl.ds(i * L, L)] = jnp.minimum(c16, N - 1)
            return 0
        lax.fori_loop(0, GROUPS, clamp, 0)
        pltpu.async_copy(dinvh.at[cbuf], dbuf, sem).wait()

        def mul(i, _):
            ebuf[pl.ds(i * L, L)] = ebuf[pl.ds(i * L, L)] * dbuf[pl.ds(i * L, L)]
            return 0
        lax.fori_loop(0, GROUPS, mul, 0)
        pltpu.sync_copy(ebuf, wcolh.at[pl.ds(base, C)])
        return 0
    lax.fori_loop(0, NCHUNK3, chunk, 0)


_wcol_call = pl.kernel(
    _wcol_kernel,
    out_type=jax.ShapeDtypeStruct((EPAD,), jnp.float32),
    mesh=_mesh,
    scratch_types=[
        pltpu.VMEM((C,), jnp.int32),
        pltpu.VMEM((C,), jnp.float32),
        pltpu.VMEM((C,), jnp.float32),
        pltpu.SemaphoreType.DMA,
    ],
)


def _dinv_kernel(deg_ref, out_ref):
    out_ref[...] = lax.rsqrt(deg_ref[...] + 1.0)


def _dinv_call(deg2d):
    return pl.pallas_call(
        _dinv_kernel,
        out_shape=jax.ShapeDtypeStruct((RS, 32), jnp.float32),
    )(deg2d)


def _xw0_kernel(x_ref, w_ref, dinv_ref, out_ref):
    out_ref[:, 0:HID] = jnp.dot(x_ref[...], w_ref[...],
                                preferred_element_type=jnp.float32, precision=lax.Precision.HIGHEST) * dinv_ref[...]


def _xw0_call(x, W, dinv_c):
    grid = N // RB4
    return pl.pallas_call(
        _xw0_kernel,
        grid=(grid,),
        in_specs=[
            pl.BlockSpec((RB4, FIN), lambda i: (i, 0)),
            pl.BlockSpec((FIN, HID), lambda i: (0, 0)),
            pl.BlockSpec((RB4, 1), lambda i: (i, 0)),
        ],
        out_specs=pl.BlockSpec((RB4, XW), lambda i: (i, 0)),
        out_shape=jax.ShapeDtypeStruct((N, XW), jnp.float32),
    )(x, W, dinv_c)


def _stats_kernel(msum_ref, xwp_ref, dinv_ref, b_ref, batch_ref,
                  t_ref, s1_ref, s2_ref, cnt_ref):
    i = pl.program_id(0)
    t = msum_ref[...] + xwp_ref[:, 0:HID] * dinv_ref[...] + b_ref[...]
    t_ref[...] = t
    seg = lax.broadcasted_iota(jnp.int32, (1, NG), 1)
    oh = (batch_ref[...] == seg).astype(jnp.float32)
    dn = (((0,), (0,)), ((), ()))
    s1 = lax.dot_general(oh, t, dn, preferred_element_type=jnp.float32, precision=lax.Precision.HIGHEST)
    s2 = lax.dot_general(oh, t * t, dn, preferred_element_type=jnp.float32, precision=lax.Precision.HIGHEST)
    cnt = lax.dot_general(oh, jnp.ones((RB, 1), jnp.float32), dn,
                          preferred_element_type=jnp.float32, precision=lax.Precision.HIGHEST)

    @pl.when(i == 0)
    def _():
        s1_ref[...] = jnp.zeros_like(s1_ref)
        s2_ref[...] = jnp.zeros_like(s2_ref)
        cnt_ref[...] = jnp.zeros_like(cnt_ref)

    s1_ref[...] += s1
    s2_ref[...] += s2
    cnt_ref[...] += cnt


def _stats_call(msum, xwp, dinv_c, b_row, batch2d):
    grid = N // RB
    return pl.pallas_call(
        _stats_kernel,
        grid=(grid,),
        in_specs=[
            pl.BlockSpec((RB, HID), lambda i: (i, 0)),
            pl.BlockSpec((RB, XW), lambda i: (i, 0)),
            pl.BlockSpec((RB, 1), lambda i: (i, 0)),
            pl.BlockSpec((1, HID), lambda i: (0, 0)),
            pl.BlockSpec((RB, 1), lambda i: (i, 0)),
        ],
        out_specs=[
            pl.BlockSpec((RB, HID), lambda i: (i, 0)),
            pl.BlockSpec((NG, HID), lambda i: (0, 0)),
            pl.BlockSpec((NG, HID), lambda i: (0, 0)),
            pl.BlockSpec((NG, 1), lambda i: (0, 0)),
        ],
        out_shape=[
            jax.ShapeDtypeStruct((N, HID), jnp.float32),
            jax.ShapeDtypeStruct((NG, HID), jnp.float32),
            jax.ShapeDtypeStruct((NG, HID), jnp.float32),
            jax.ShapeDtypeStruct((NG, 1), jnp.float32),
        ],
    )(msum, xwp, dinv_c, b_row, batch2d)


def _norm_kernel(t_ref, batch_ref, s1_ref, s2_ref, cnt_ref, gw_ref, gb_ref,
                 ms_ref, w_ref, dinv_ref, out_ref, *, do_relu, do_matmul):
    c = jnp.maximum(cnt_ref[...], 1.0)
    m = s1_ref[...] / c
    q = s2_ref[...] / c
    ms = ms_ref[...]
    var = q - m * m * ms * (2.0 - ms)
    invstd = lax.rsqrt(var + 1e-5)
    mm = m * ms
    seg = lax.broadcasted_iota(jnp.int32, (1, NG), 1)
    oh = (batch_ref[...] == seg).astype(jnp.float32)
    mm_r = jnp.dot(oh, mm, preferred_element_type=jnp.float32, precision=lax.Precision.HIGHEST)
    is_r = jnp.dot(oh, invstd, preferred_element_type=jnp.float32, precision=lax.Precision.HIGHEST)
    y = gw_ref[...] * (t_ref[...] - mm_r) * is_r + gb_ref[...]
    if do_relu:
        y = jnp.maximum(y, 0.0)
    if do_matmul:
        y = jnp.dot(y, w_ref[...],
                    preferred_element_type=jnp.float32, precision=lax.Precision.HIGHEST) * dinv_ref[...]
        out_ref[:, 0:HID] = y
    else:
        out_ref[...] = y


def _norm_call(t, batch2d, s1, s2, cnt, gw, gb, ms, W, dinv_c,
               do_relu, do_matmul):
    grid = N // RB
    return pl.pallas_call(
        functools.partial(_norm_kernel, do_relu=do_relu, do_matmul=do_matmul),
        grid=(grid,),
        in_specs=[
            pl.BlockSpec((RB, HID), lambda i: (i, 0)),
            pl.BlockSpec((RB, 1), lambda i: (i, 0)),
            pl.BlockSpec((NG, HID), lambda i: (0, 0)),
            pl.BlockSpec((NG, HID), lambda i: (0, 0)),
            pl.BlockSpec((NG, 1), lambda i: (0, 0)),
            pl.BlockSpec((1, HID), lambda i: (0, 0)),
            pl.BlockSpec((1, HID), lambda i: (0, 0)),
            pl.BlockSpec((1, HID), lambda i: (0, 0)),
            pl.BlockSpec((HID, HID), lambda i: (0, 0)),
            pl.BlockSpec((RB, 1), lambda i: (i, 0)),
        ],
        out_specs=pl.BlockSpec((RB, XW if do_matmul else HID),
                               lambda i: (i, 0)),
        out_shape=jax.ShapeDtypeStruct((N, XW if do_matmul else HID),
                                       jnp.float32),
    )(t, batch2d, s1, s2, cnt, gw, gb, ms, W, dinv_c)


def _gru_kernel(seq_ref, wi0_ref, wh0_ref, bi0_ref, bh0_ref,
                wi1_ref, wh1_ref, bi1_ref, bh1_ref, wr_ref, br_ref,
                out_ref, h1_ref, h2_ref):
    B = NB * (BL // T)
    h1_ref[...] = jnp.zeros((B, GRU_H), jnp.float32)
    h2_ref[...] = jnp.zeros((B, GRU_H), jnp.float32)

    def gru_step(xt, h, wi, wh, bi, bh):
        gi = jnp.dot(xt, wi, preferred_element_type=jnp.float32, precision=lax.Precision.HIGHEST) + bi
        gh = jnp.dot(h, wh, preferred_element_type=jnp.float32, precision=lax.Precision.HIGHEST) + bh
        r = jax.nn.sigmoid(gi[:, 0:GRU_H] + gh[:, 0:GRU_H])
        z = jax.nn.sigmoid(gi[:, GRU_H:2 * GRU_H] + gh[:, GRU_H:2 * GRU_H])
        n = jnp.tanh(gi[:, 2 * GRU_H:] + r * gh[:, 2 * GRU_H:])
        return (1.0 - z) * n + z * h

    def step(tt, _):
        xt = seq_ref[tt]
        h1 = gru_step(xt, h1_ref[...], wi0_ref[...], wh0_ref[...],
                      bi0_ref[...], bh0_ref[...])
        h1_ref[...] = h1
        h2 = gru_step(h1, h2_ref[...], wi1_ref[...], wh1_ref[...],
                      bi1_ref[...], bh1_ref[...])
        h2_ref[...] = h2
        return 0

    lax.fori_loop(0, T, step, 0)
    out_ref[...] = jnp.dot(h2_ref[...], wr_ref[...],
                           preferred_element_type=jnp.float32, precision=lax.Precision.HIGHEST) + br_ref[...]


def _gru_call(seq, wi0, wh0, bi0, bh0, wi1, wh1, bi1, bh1, wrp, br128):
    B = NB * (BL // T)
    return pl.pallas_call(
        _gru_kernel,
        out_shape=jax.ShapeDtypeStruct((B, 128), jnp.float32),
        scratch_shapes=[
            pltpu.VMEM((B, GRU_H), jnp.float32),
            pltpu.VMEM((B, GRU_H), jnp.float32),
        ],
    )(seq, wi0, wh0, bi0, bh0, wi1, wh1, bi1, bh1, wrp, br128)


def kernel(x, edge_index, edge_attr, batch, block_lengths, params):
    row = edge_index[0]
    col = edge_index[1]
    pad = EPAD - E
    rowp = jnp.concatenate([row, jnp.zeros((pad,), row.dtype)])
    colp = jnp.concatenate([col, jnp.full((pad,), N, col.dtype)])
    ewp = jnp.concatenate([edge_attr, jnp.zeros((pad,), edge_attr.dtype)])

    degp = _deg_call(colp, ewp)
    deg2d = degp[:, :HALF].reshape(RS, 32)
    dinv2d = _dinv_call(deg2d)
    dinv = dinv2d.reshape(N)
    dinv_c = dinv.reshape(N, 1)
    wcol = _wcol_call(colp, ewp, dinv)
    batch2d = batch.reshape(N, 1)

    xwp = _xw0_call(x, params['W0'], dinv_c)
    h = None
    for li in range(5):
        msgs = xwp[rowp][:, :HID] * wcol[:, None]
        msum = jnp.zeros((N, HID), jnp.float32).at[colp].add(msgs)
        t, s1, s2, cnt = _stats_call(msum, xwp, dinv_c,
                                     params['b%d' % li].reshape(1, HID),
                                     batch2d)
        gw = params['gn_w%d' % li].reshape(1, HID)
        gb = params['gn_b%d' % li].reshape(1, HID)
        ms = params['gn_ms%d' % li].reshape(1, HID)
        if li < 4:
            xwp = _norm_call(t, batch2d, s1, s2, cnt, gw, gb, ms,
                             params['W%d' % (li + 1)], dinv_c,
                             do_relu=True, do_matmul=True)
        else:
            h = _norm_call(t, batch2d, s1, s2, cnt, gw, gb, ms,
                           params['W4'], dinv_c,
                           do_relu=False, do_matmul=False)

    n_per = BL // T
    Hm = h.reshape(NB, T, n_per, GOUT).transpose(0, 2, 1, 3)
    seq = Hm.reshape(NB * n_per, T, GOUT).transpose(1, 0, 2)

    wrp = jnp.pad(params['Wr'].T, ((0, 0), (0, 127)))
    br128 = jnp.broadcast_to(params['br'].reshape(1, 1), (1, 128))
    out128 = _gru_call(
        seq,
        params['Wih0'].T, params['Whh0'].T,
        params['bih0'].reshape(1, 3 * GRU_H), params['bhh0'].reshape(1, 3 * GRU_H),
        params['Wih1'].T, params['Whh1'].T,
        params['bih1'].reshape(1, 3 * GRU_H), params['bhh1'].reshape(1, 3 * GRU_H),
        wrp, br128)
    return out128[:, :1]
